# baseline (speedup 1.0000x reference)
import functools

import jax
import jax.numpy as jnp
from jax import lax
from jax.experimental import pallas as pl
from jax.experimental.pallas import tpu as pltpu
from jax.experimental.pallas import tpu_sc as plsc

NC, NS, L = 2, 16, 16
NW = NC * NS
B, S, D, H, C = 4096, 200, 64, 128, 4
BPW = B // NW
ABLK = BPW // 8
KD = D // L
ROWS_PER_ITER = 8
NPHASE = 4
REM = S - 128


def _pool_sc_kernel(idx_hbm, table_hbm, out_hbm, idx_v, buf_a, buf_b, out_v,
                    sem_a, sem_b):
    wid = lax.axis_index("s") * NC + lax.axis_index("c")

    pltpu.sync_copy(idx_hbm.at[pl.ds(wid * ABLK, ABLK)], idx_v)

    def issue(row, buf, sem):
        a = row >> 3
        b = row & 7
        pltpu.async_copy(table_hbm.at[idx_v.at[a, 0, b]],
                         buf.at[pl.ds(0, 128)], sem)
        pltpu.async_copy(table_hbm.at[idx_v.at[a, 1, b, pl.ds(0, REM)]],
                         buf.at[pl.ds(128, REM)], sem)

    def drain(row, buf, sem):
        a = row >> 3
        b = row & 7
        pltpu.make_async_copy(table_hbm.at[idx_v.at[a, 0, b]],
                              buf.at[pl.ds(0, 128)], sem).wait()
        pltpu.make_async_copy(table_hbm.at[idx_v.at[a, 1, b, pl.ds(0, REM)]],
                              buf.at[pl.ds(128, REM)], sem).wait()

    def accumulate(row, buf):
        zero = jnp.zeros((L,), jnp.float32)
        init = tuple(zero for _ in range(NPHASE * KD))

        def body(i, accs):
            accs = list(accs)
            base = i * ROWS_PER_ITER
            for u in range(ROWS_PER_ITER):
                p = u % NPHASE
                for k in range(KD):
                    v = buf[base + u, pl.ds(k * L, L)]
                    accs[p * KD + k] = accs[p * KD + k] + v
            return tuple(accs)

        accs = lax.fori_loop(0, S // ROWS_PER_ITER, body, init)
        for k in range(KD):
            tot = accs[k]
            for p in range(1, NPHASE):
                tot = tot + accs[p * KD + k]
            out_v[row, pl.ds(k * L, L)] = tot

    issue(0, buf_a, sem_a)

    def row_pair(rr, _):
        r0 = rr * 2
        issue(r0 + 1, buf_b, sem_b)
        drain(r0, buf_a, sem_a)
        accumulate(r0, buf_a)

        @pl.when(r0 + 2 < BPW)
        def _():
            issue(r0 + 2, buf_a, sem_a)

        drain(r0 + 1, buf_b, sem_b)
        accumulate(r0 + 1, buf_b)
        return 0

    lax.fori_loop(0, BPW // 2, row_pair, 0)

    pltpu.sync_copy(out_v, out_hbm.at[pl.ds(wid * BPW, BPW)])


@functools.partial(
    pl.kernel,
    out_type=jax.ShapeDtypeStruct((B, D), jnp.float32),
    mesh=plsc.VectorSubcoreMesh(core_axis_name="c", subcore_axis_name="s",
                                num_cores=NC, num_subcores=NS),
    scratch_types=[
        pltpu.VMEM((ABLK, 2, 8, 128), jnp.int32),
        pltpu.VMEM((S, D), jnp.float32),
        pltpu.VMEM((S, D), jnp.float32),
        pltpu.VMEM((BPW, D), jnp.float32),
        pltpu.SemaphoreType.DMA,
        pltpu.SemaphoreType.DMA,
    ],
    compiler_params=pltpu.CompilerParams(use_tc_tiling_on_sc=False),
)
def _pool_sc(idx_hbm, table_hbm, out_hbm, idx_v, buf_a, buf_b, out_v,
             sem_a, sem_b):
    _pool_sc_kernel(idx_hbm, table_hbm, out_hbm, idx_v, buf_a, buf_b, out_v,
                    sem_a, sem_b)


def _mlp_body(x_ref, w1_ref, b1_ref, w2_ref, b2_ref, o_ref):
    x = x_ref[...]
    h = jnp.dot(x, w1_ref[...], preferred_element_type=jnp.float32)
    h = jnp.maximum(h + b1_ref[...], 0.0)
    o = jnp.dot(h, w2_ref[...], preferred_element_type=jnp.float32)
    o_ref[...] = o + b2_ref[...]


def _mlp_tc(x, w1_scaled, b1, w2, b2):
    return pl.pallas_call(
        _mlp_body,
        out_shape=jax.ShapeDtypeStruct((B, C), jnp.float32),
    )(x, w1_scaled, b1.reshape(1, H), w2, b2.reshape(1, C))


def kernel(reports, table, W1, b1, W2, b2):
    rp = jnp.pad(reports, ((0, 0), (0, 256 - S)))
    idx4 = rp.reshape(B // 8, 8, 2, 128).transpose(0, 2, 1, 3)
    pooled_sums = _pool_sc(idx4, table)
    return _mlp_tc(pooled_sums, W1 * (1.0 / S), b1, W2, b2)


# speedup vs baseline: 1.0021x; 1.0021x over previous
import functools

import jax
import jax.numpy as jnp
from jax import lax
from jax.experimental import pallas as pl
from jax.experimental.pallas import tpu as pltpu
from jax.experimental.pallas import tpu_sc as plsc

NC, NS, L = 2, 16, 16
NW = NC * NS
B, S, D, H, C = 4096, 200, 64, 128, 4
BPW = B // NW
ABLK = BPW // 8
KD = D // L
ROWS_PER_ITER = 8
NPHASE = 4
REM = S - 128


def _pool_sc_kernel(idx_hbm, table_hbm, out_hbm, idx_v, buf_a, buf_b, out_v,
                    sem_a, sem_b):
    wid = lax.axis_index("s") * NC + lax.axis_index("c")

    pltpu.sync_copy(idx_hbm.at[pl.ds(wid * BPW, BPW)], idx_v)

    def issue(row, buf, sem):
        pltpu.async_copy(table_hbm.at[idx_v.at[row, pl.ds(0, 128)]],
                         buf.at[pl.ds(0, 128)], sem)
        pltpu.async_copy(table_hbm.at[idx_v.at[row, pl.ds(128, REM)]],
                         buf.at[pl.ds(128, REM)], sem)

    def drain(row, buf, sem):
        pltpu.make_async_copy(table_hbm.at[idx_v.at[row, pl.ds(0, 128)]],
                              buf.at[pl.ds(0, 128)], sem).wait()
        pltpu.make_async_copy(table_hbm.at[idx_v.at[row, pl.ds(128, REM)]],
                              buf.at[pl.ds(128, REM)], sem).wait()

    def accumulate(row, buf):
        zero = jnp.zeros((L,), jnp.float32)
        init = tuple(zero for _ in range(NPHASE * KD))

        def body(i, accs):
            accs = list(accs)
            base = i * ROWS_PER_ITER
            for u in range(ROWS_PER_ITER):
                p = u % NPHASE
                for k in range(KD):
                    v = buf[base + u, pl.ds(k * L, L)]
                    accs[p * KD + k] = accs[p * KD + k] + v
            return tuple(accs)

        accs = lax.fori_loop(0, S // ROWS_PER_ITER, body, init)
        for k in range(KD):
            tot = accs[k]
            for p in range(1, NPHASE):
                tot = tot + accs[p * KD + k]
            out_v[row, pl.ds(k * L, L)] = tot

    issue(0, buf_a, sem_a)

    def row_pair(rr, _):
        r0 = rr * 2
        issue(r0 + 1, buf_b, sem_b)
        drain(r0, buf_a, sem_a)
        accumulate(r0, buf_a)

        @pl.when(r0 + 2 < BPW)
        def _():
            issue(r0 + 2, buf_a, sem_a)

        drain(r0 + 1, buf_b, sem_b)
        accumulate(r0 + 1, buf_b)
        return 0

    lax.fori_loop(0, BPW // 2, row_pair, 0)

    pltpu.sync_copy(out_v, out_hbm.at[pl.ds(wid * BPW, BPW)])


@functools.partial(
    pl.kernel,
    out_type=jax.ShapeDtypeStruct((B, D), jnp.float32),
    mesh=plsc.VectorSubcoreMesh(core_axis_name="c", subcore_axis_name="s",
                                num_cores=NC, num_subcores=NS),
    scratch_types=[
        pltpu.VMEM((BPW, 256), jnp.int32),
        pltpu.VMEM((S, D), jnp.float32),
        pltpu.VMEM((S, D), jnp.float32),
        pltpu.VMEM((BPW, D), jnp.float32),
        pltpu.SemaphoreType.DMA,
        pltpu.SemaphoreType.DMA,
    ],
    compiler_params=pltpu.CompilerParams(use_tc_tiling_on_sc=False),
)
def _pool_sc(idx_hbm, table_hbm, out_hbm, idx_v, buf_a, buf_b, out_v,
             sem_a, sem_b):
    _pool_sc_kernel(idx_hbm, table_hbm, out_hbm, idx_v, buf_a, buf_b, out_v,
                    sem_a, sem_b)


def _mlp_body(x_ref, w1_ref, b1_ref, w2_ref, b2_ref, o_ref):
    x = x_ref[...]
    h = jnp.dot(x, w1_ref[...], preferred_element_type=jnp.float32)
    h = jnp.maximum(h + b1_ref[...], 0.0)
    o = jnp.dot(h, w2_ref[...], preferred_element_type=jnp.float32)
    o_ref[...] = o + b2_ref[...]


def _mlp_tc(x, w1_scaled, b1, w2, b2):
    return pl.pallas_call(
        _mlp_body,
        out_shape=jax.ShapeDtypeStruct((B, C), jnp.float32),
    )(x, w1_scaled, b1.reshape(1, H), w2, b2.reshape(1, C))


def kernel(reports, table, W1, b1, W2, b2):
    rp = jnp.pad(reports, ((0, 0), (0, 256 - S)))
    pooled_sums = _pool_sc(rp, table)
    return _mlp_tc(pooled_sums, W1 * (1.0 / S), b1, W2, b2)


# speedup vs baseline: 1.5344x; 1.5312x over previous
import functools

import jax
import jax.numpy as jnp
from jax import lax
from jax.experimental import pallas as pl
from jax.experimental.pallas import tpu as pltpu
from jax.experimental.pallas import tpu_sc as plsc

NC, NS, L = 2, 16, 16
NW = NC * NS
B, S, D, H, C = 4096, 200, 64, 128, 4
V = 1000000
BPW = B // NW
KD = H // L
ROWS_PER_ITER = 4
NPHASE = 4
REM = S - 128
VBLK = 8192


def _expand_body(xt_ref, w1_ref, g_ref):
    g_ref[...] = lax.dot_general(
        xt_ref[...], w1_ref[...], (((0,), (0,)), ((), ())),
        preferred_element_type=jnp.float32)


def _expand_tc(table_t, w1):
    return pl.pallas_call(
        _expand_body,
        grid=((V + VBLK - 1) // VBLK,),
        in_specs=[
            pl.BlockSpec((D, VBLK), lambda i: (0, i)),
            pl.BlockSpec((D, H), lambda i: (0, 0)),
        ],
        out_specs=pl.BlockSpec((VBLK, H), lambda i: (i, 0)),
        out_shape=jax.ShapeDtypeStruct((V, H), jnp.float32),
    )(table_t, w1)


def _pool_sc_kernel(idx_hbm, g_hbm, out_hbm, idx_v, buf_a, buf_b, out_v,
                    sem_a, sem_b):
    wid = lax.axis_index("s") * NC + lax.axis_index("c")

    pltpu.sync_copy(idx_hbm.at[pl.ds(wid * BPW, BPW)], idx_v)

    def issue(row, buf, sem):
        pltpu.async_copy(g_hbm.at[idx_v.at[row, pl.ds(0, 128)]],
                         buf.at[pl.ds(0, 128)], sem)
        pltpu.async_copy(g_hbm.at[idx_v.at[row, pl.ds(128, REM)]],
                         buf.at[pl.ds(128, REM)], sem)

    def drain(row, buf, sem):
        pltpu.make_async_copy(g_hbm.at[idx_v.at[row, pl.ds(0, 128)]],
                              buf.at[pl.ds(0, 128)], sem).wait()
        pltpu.make_async_copy(g_hbm.at[idx_v.at[row, pl.ds(128, REM)]],
                              buf.at[pl.ds(128, REM)], sem).wait()

    def accumulate(row, buf):
        zero = jnp.zeros((L,), jnp.float32)
        init = tuple(zero for _ in range(NPHASE * KD))

        def body(i, accs):
            accs = list(accs)
            base = i * ROWS_PER_ITER
            for u in range(ROWS_PER_ITER):
                p = u % NPHASE
                for k in range(KD):
                    v = buf[base + u, pl.ds(k * L, L)]
                    accs[p * KD + k] = accs[p * KD + k] + v
            return tuple(accs)

        accs = lax.fori_loop(0, S // ROWS_PER_ITER, body, init)
        for k in range(KD):
            tot = accs[k]
            for p in range(1, NPHASE):
                tot = tot + accs[p * KD + k]
            out_v[row, pl.ds(k * L, L)] = tot

    issue(0, buf_a, sem_a)

    def row_pair(rr, _):
        r0 = rr * 2
        issue(r0 + 1, buf_b, sem_b)
        drain(r0, buf_a, sem_a)
        accumulate(r0, buf_a)

        @pl.when(r0 + 2 < BPW)
        def _():
            issue(r0 + 2, buf_a, sem_a)

        drain(r0 + 1, buf_b, sem_b)
        accumulate(r0 + 1, buf_b)
        return 0

    lax.fori_loop(0, BPW // 2, row_pair, 0)

    pltpu.sync_copy(out_v, out_hbm.at[pl.ds(wid * BPW, BPW)])


@functools.partial(
    pl.kernel,
    out_type=jax.ShapeDtypeStruct((B, H), jnp.float32),
    mesh=plsc.VectorSubcoreMesh(core_axis_name="c", subcore_axis_name="s",
                                num_cores=NC, num_subcores=NS),
    scratch_types=[
        pltpu.VMEM((BPW, S), jnp.int32),
        pltpu.VMEM((S, H), jnp.float32),
        pltpu.VMEM((S, H), jnp.float32),
        pltpu.VMEM((BPW, H), jnp.float32),
        pltpu.SemaphoreType.DMA,
        pltpu.SemaphoreType.DMA,
    ],
    compiler_params=pltpu.CompilerParams(use_tc_tiling_on_sc=True),
)
def _pool_sc(idx_hbm, g_hbm, out_hbm, idx_v, buf_a, buf_b, out_v,
             sem_a, sem_b):
    _pool_sc_kernel(idx_hbm, g_hbm, out_hbm, idx_v, buf_a, buf_b, out_v,
                    sem_a, sem_b)


def _epilogue_body(p_ref, b1_ref, w2_ref, b2_ref, o_ref):
    h = jnp.maximum(p_ref[...] * (1.0 / S) + b1_ref[...], 0.0)
    o = jnp.dot(h, w2_ref[...], preferred_element_type=jnp.float32)
    o_ref[...] = o + b2_ref[...]


def _epilogue_tc(pooled, b1, w2, b2):
    return pl.pallas_call(
        _epilogue_body,
        out_shape=jax.ShapeDtypeStruct((B, C), jnp.float32),
    )(pooled, b1.reshape(1, H), w2, b2.reshape(1, C))


def kernel(reports, table, W1, b1, W2, b2):
    g = _expand_tc(table.T, W1)
    pooled = _pool_sc(reports, g)
    return _epilogue_tc(pooled, b1, W2, b2)


# speedup vs baseline: 1.7119x; 1.1157x over previous
import functools

import jax
import jax.numpy as jnp
from jax import lax
from jax.experimental import pallas as pl
from jax.experimental.pallas import tpu as pltpu
from jax.experimental.pallas import tpu_sc as plsc

NC, NS, L = 2, 16, 16
NW = NC * NS
B, S, D, H, C = 4096, 200, 64, 128, 4
V = 1000000
BPW = B // NW
KD = H // L
ROWS_PER_ITER = 4
NPHASE = 4
REM = S - 128
VBLK = 16384


def _expand_body(xt_ref, w1_ref, g_ref):
    g_ref[...] = lax.dot_general(
        xt_ref[...], w1_ref[...], (((0,), (0,)), ((), ())),
        preferred_element_type=jnp.float32)


def _expand_tc(table_t, w1):
    return pl.pallas_call(
        _expand_body,
        grid=((V + VBLK - 1) // VBLK,),
        in_specs=[
            pl.BlockSpec((D, VBLK), lambda i: (0, i)),
            pl.BlockSpec((D, H), lambda i: (0, 0)),
        ],
        out_specs=pl.BlockSpec((VBLK, H), lambda i: (i, 0)),
        out_shape=jax.ShapeDtypeStruct((V, H), jnp.float32),
    )(table_t, w1)


CHK = (128, 72)
NCHUNK = 2 * BPW
NBUF = 4


def _pool_sc_kernel(idx_hbm, g_hbm, out_hbm, idx_v, buf0, buf1, buf2, buf3,
                    out_v, sem0, sem1, sem2, sem3):
    wid = lax.axis_index("s") * NC + lax.axis_index("c")
    bufs = (buf0, buf1, buf2, buf3)
    sems = (sem0, sem1, sem2, sem3)

    pltpu.sync_copy(idx_hbm.at[pl.ds(wid * BPW, BPW)], idx_v)

    def src(c, par):
        return g_hbm.at[idx_v.at[c >> 1, pl.ds(par * CHK[0], CHK[par])]]

    def issue(c, par, j):
        pltpu.async_copy(src(c, par), bufs[j].at[pl.ds(0, CHK[par])], sems[j])

    def drain(c, par, j):
        pltpu.make_async_copy(src(c, par), bufs[j].at[pl.ds(0, CHK[par])],
                              sems[j]).wait()

    def acc_chunk(par, buf, accs):
        def body(i, accs):
            accs = list(accs)
            base = i * ROWS_PER_ITER
            for u in range(ROWS_PER_ITER):
                p = u % NPHASE
                for k in range(KD):
                    v = buf[base + u, pl.ds(k * L, L)]
                    accs[p * KD + k] = accs[p * KD + k] + v
            return tuple(accs)

        return lax.fori_loop(0, CHK[par] // ROWS_PER_ITER, body, accs)

    def finalize(row, accs):
        for k in range(KD):
            tot = accs[k]
            for p in range(1, NPHASE):
                tot = tot + accs[p * KD + k]
            out_v[row, pl.ds(k * L, L)] = tot

    zero = jnp.zeros((L,), jnp.float32)
    init = tuple(zero for _ in range(NPHASE * KD))

    for j in range(NBUF):
        issue(j, j & 1, j)

    def ring_iter(rr, _):
        c0 = rr * NBUF
        for h in range(2):
            accs = init
            for par in range(2):
                j = h * 2 + par
                c = c0 + j
                drain(c, par, j)
                accs = acc_chunk(par, bufs[j], accs)

                @pl.when(c + NBUF < NCHUNK)
                def _():
                    issue(c + NBUF, par, j)
            finalize(c0 // 2 + h, accs)
        return 0

    lax.fori_loop(0, NCHUNK // NBUF, ring_iter, 0)

    pltpu.sync_copy(out_v, out_hbm.at[pl.ds(wid * BPW, BPW)])


@functools.partial(
    pl.kernel,
    out_type=jax.ShapeDtypeStruct((B, H), jnp.float32),
    mesh=plsc.VectorSubcoreMesh(core_axis_name="c", subcore_axis_name="s",
                                num_cores=NC, num_subcores=NS),
    scratch_types=[
        pltpu.VMEM((BPW, S), jnp.int32),
        pltpu.VMEM((CHK[0], H), jnp.float32),
        pltpu.VMEM((CHK[0], H), jnp.float32),
        pltpu.VMEM((CHK[0], H), jnp.float32),
        pltpu.VMEM((CHK[0], H), jnp.float32),
        pltpu.VMEM((BPW, H), jnp.float32),
        pltpu.SemaphoreType.DMA,
        pltpu.SemaphoreType.DMA,
        pltpu.SemaphoreType.DMA,
        pltpu.SemaphoreType.DMA,
    ],
    compiler_params=pltpu.CompilerParams(use_tc_tiling_on_sc=True),
)
def _pool_sc(idx_hbm, g_hbm, out_hbm, idx_v, buf0, buf1, buf2, buf3, out_v,
             sem0, sem1, sem2, sem3):
    _pool_sc_kernel(idx_hbm, g_hbm, out_hbm, idx_v, buf0, buf1, buf2, buf3,
                    out_v, sem0, sem1, sem2, sem3)


def _epilogue_body(p_ref, b1_ref, w2_ref, b2_ref, o_ref):
    h = jnp.maximum(p_ref[...] * (1.0 / S) + b1_ref[...], 0.0)
    o = jnp.dot(h, w2_ref[...], preferred_element_type=jnp.float32)
    o_ref[...] = o + b2_ref[...]


def _epilogue_tc(pooled, b1, w2, b2):
    return pl.pallas_call(
        _epilogue_body,
        out_shape=jax.ShapeDtypeStruct((B, C), jnp.float32),
    )(pooled, b1.reshape(1, H), w2, b2.reshape(1, C))


def kernel(reports, table, W1, b1, W2, b2):
    g = _expand_tc(table.T, W1)
    pooled = _pool_sc(reports, g)
    return _epilogue_tc(pooled, b1, W2, b2)


# speedup vs baseline: 1.7274x; 1.0091x over previous
import functools

import jax
import jax.numpy as jnp
from jax import lax
from jax.experimental import pallas as pl
from jax.experimental.pallas import tpu as pltpu
from jax.experimental.pallas import tpu_sc as plsc

NC, NS, L = 2, 16, 16
NW = NC * NS
B, S, D, H, C = 4096, 200, 64, 128, 4
V = 1000000
BPW = B // NW
KD = H // L
ROWS_PER_ITER = 8
NPHASE = 4
REM = S - 128
VBLK = 32768


def _expand_body(xt_ref, w1_ref, g_ref):
    g_ref[...] = lax.dot_general(
        xt_ref[...], w1_ref[...], (((0,), (0,)), ((), ())),
        preferred_element_type=jnp.float32)


def _expand_tc(table_t, w1):
    return pl.pallas_call(
        _expand_body,
        grid=((V + VBLK - 1) // VBLK,),
        in_specs=[
            pl.BlockSpec((D, VBLK), lambda i: (0, i)),
            pl.BlockSpec((D, H), lambda i: (0, 0)),
        ],
        out_specs=pl.BlockSpec((VBLK, H), lambda i: (i, 0)),
        out_shape=jax.ShapeDtypeStruct((V, H), jnp.float32),
    )(table_t, w1)


CHK = (128, 72)
NCHUNK = 2 * BPW
NBUF = 4


def _pool_sc_kernel(idx_hbm, g_hbm, out_hbm, idx_v, buf0, buf1, buf2, buf3,
                    out_v, sem0, sem1, sem2, sem3):
    wid = lax.axis_index("s") * NC + lax.axis_index("c")
    bufs = (buf0, buf1, buf2, buf3)
    sems = (sem0, sem1, sem2, sem3)

    pltpu.sync_copy(idx_hbm.at[pl.ds(wid * BPW, BPW)], idx_v)

    def src(c, par):
        return g_hbm.at[idx_v.at[c >> 1, pl.ds(par * CHK[0], CHK[par])]]

    def issue(c, par, j):
        pltpu.async_copy(src(c, par), bufs[j].at[pl.ds(0, CHK[par])], sems[j])

    def drain(c, par, j):
        pltpu.make_async_copy(src(c, par), bufs[j].at[pl.ds(0, CHK[par])],
                              sems[j]).wait()

    def acc_chunk(par, buf, accs):
        def body(i, accs):
            accs = list(accs)
            base = i * ROWS_PER_ITER
            for u in range(ROWS_PER_ITER):
                p = u % NPHASE
                for k in range(KD):
                    v = buf[base + u, pl.ds(k * L, L)]
                    accs[p * KD + k] = accs[p * KD + k] + v
            return tuple(accs)

        return lax.fori_loop(0, CHK[par] // ROWS_PER_ITER, body, accs)

    def finalize(row, accs):
        for k in range(KD):
            tot = accs[k]
            for p in range(1, NPHASE):
                tot = tot + accs[p * KD + k]
            out_v[row, pl.ds(k * L, L)] = tot

    zero = jnp.zeros((L,), jnp.float32)
    init = tuple(zero for _ in range(NPHASE * KD))

    for j in range(NBUF):
        issue(j, j & 1, j)

    def ring_iter(rr, _):
        c0 = rr * NBUF
        for h in range(2):
            accs = init
            for par in range(2):
                j = h * 2 + par
                c = c0 + j
                drain(c, par, j)
                accs = acc_chunk(par, bufs[j], accs)

                @pl.when(c + NBUF < NCHUNK)
                def _():
                    issue(c + NBUF, par, j)
            finalize(c0 // 2 + h, accs)
        return 0

    lax.fori_loop(0, NCHUNK // NBUF, ring_iter, 0)

    pltpu.sync_copy(out_v, out_hbm.at[pl.ds(wid * BPW, BPW)])


@functools.partial(
    pl.kernel,
    out_type=jax.ShapeDtypeStruct((B, H), jnp.float32),
    mesh=plsc.VectorSubcoreMesh(core_axis_name="c", subcore_axis_name="s",
                                num_cores=NC, num_subcores=NS),
    scratch_types=[
        pltpu.VMEM((BPW, S), jnp.int32),
        pltpu.VMEM((CHK[0], H), jnp.float32),
        pltpu.VMEM((CHK[0], H), jnp.float32),
        pltpu.VMEM((CHK[0], H), jnp.float32),
        pltpu.VMEM((CHK[0], H), jnp.float32),
        pltpu.VMEM((BPW, H), jnp.float32),
        pltpu.SemaphoreType.DMA,
        pltpu.SemaphoreType.DMA,
        pltpu.SemaphoreType.DMA,
        pltpu.SemaphoreType.DMA,
    ],
    compiler_params=pltpu.CompilerParams(use_tc_tiling_on_sc=True),
)
def _pool_sc(idx_hbm, g_hbm, out_hbm, idx_v, buf0, buf1, buf2, buf3, out_v,
             sem0, sem1, sem2, sem3):
    _pool_sc_kernel(idx_hbm, g_hbm, out_hbm, idx_v, buf0, buf1, buf2, buf3,
                    out_v, sem0, sem1, sem2, sem3)


def _epilogue_body(p_ref, b1_ref, w2_ref, b2_ref, o_ref):
    h = jnp.maximum(p_ref[...] * (1.0 / S) + b1_ref[...], 0.0)
    o = jnp.dot(h, w2_ref[...], preferred_element_type=jnp.float32)
    o_ref[...] = o + b2_ref[...]


def _epilogue_tc(pooled, b1, w2, b2):
    return pl.pallas_call(
        _epilogue_body,
        out_shape=jax.ShapeDtypeStruct((B, C), jnp.float32),
    )(pooled, b1.reshape(1, H), w2, b2.reshape(1, C))


def kernel(reports, table, W1, b1, W2, b2):
    g = _expand_tc(table.T, W1)
    pooled = _pool_sc(reports, g)
    return _epilogue_tc(pooled, b1, W2, b2)


# speedup vs baseline: 1.7958x; 1.0396x over previous
import functools

import jax
import jax.numpy as jnp
from jax import lax
from jax.experimental import pallas as pl
from jax.experimental.pallas import tpu as pltpu
from jax.experimental.pallas import tpu_sc as plsc

NC, NS, L = 2, 16, 16
NW = NC * NS
B, S, D, H, C = 4096, 200, 64, 128, 4
V = 1000000
BPW = B // NW
KD = H // L
ROWS_PER_ITER = 8
NPHASE = 4
REM = S - 128
VBLK = 32768


def _expand_body(xt_ref, w1_ref, g_ref):
    g_ref[...] = lax.dot_general(
        xt_ref[...], w1_ref[...], (((0,), (0,)), ((), ())),
        preferred_element_type=jnp.float32)


def _expand_tc(table_t, w1):
    return pl.pallas_call(
        _expand_body,
        grid=((V + VBLK - 1) // VBLK,),
        in_specs=[
            pl.BlockSpec((D, VBLK), lambda i: (0, i)),
            pl.BlockSpec((D, H), lambda i: (0, 0)),
        ],
        out_specs=pl.BlockSpec((VBLK, H), lambda i: (i, 0)),
        out_shape=jax.ShapeDtypeStruct((V, H), jnp.float32),
    )(table_t, w1)


CHK = (64, 64, 72)
OFF = (0, 64, 128)
NQ = len(CHK)
NBUF = 6


def _pool_sc_kernel(idx_hbm, g_hbm, out_hbm, idx_v, buf0, buf1, buf2, buf3,
                    buf4, buf5, out_v, sem0, sem1, sem2, sem3, sem4, sem5):
    wid = lax.axis_index("s") * NC + lax.axis_index("c")
    bufs = (buf0, buf1, buf2, buf3, buf4, buf5)
    sems = (sem0, sem1, sem2, sem3, sem4, sem5)

    pltpu.sync_copy(idx_hbm.at[pl.ds(wid * BPW, BPW)], idx_v)

    def src(row, q):
        return g_hbm.at[idx_v.at[row, pl.ds(OFF[q], CHK[q])]]

    def issue(row, q, j):
        pltpu.async_copy(src(row, q), bufs[j].at[pl.ds(0, CHK[q])], sems[j])

    def drain(row, q, j):
        pltpu.make_async_copy(src(row, q), bufs[j].at[pl.ds(0, CHK[q])],
                              sems[j]).wait()

    def acc_chunk(q, buf, accs):
        def body(i, accs):
            accs = list(accs)
            base = i * ROWS_PER_ITER
            for u in range(ROWS_PER_ITER):
                p = u % NPHASE
                for k in range(KD):
                    v = buf[base + u, pl.ds(k * L, L)]
                    accs[p * KD + k] = accs[p * KD + k] + v
            return tuple(accs)

        return lax.fori_loop(0, CHK[q] // ROWS_PER_ITER, body, accs)

    def finalize(row, accs):
        for k in range(KD):
            tot = accs[k]
            for p in range(1, NPHASE):
                tot = tot + accs[p * KD + k]
            out_v[row, pl.ds(k * L, L)] = tot

    zero = jnp.zeros((L,), jnp.float32)
    init = tuple(zero for _ in range(NPHASE * KD))

    for j in range(NBUF):
        issue(j // NQ, j % NQ, j)

    def ring_iter(rr, _):
        for h in range(2):
            row = rr * 2 + h
            accs = init
            for q in range(NQ):
                j = h * NQ + q
                drain(row, q, j)
                accs = acc_chunk(q, bufs[j], accs)

                @pl.when(row + 2 < BPW)
                def _():
                    issue(row + 2, q, j)
            finalize(row, accs)
        return 0

    lax.fori_loop(0, BPW // 2, ring_iter, 0)

    pltpu.sync_copy(out_v, out_hbm.at[pl.ds(wid * BPW, BPW)])


@functools.partial(
    pl.kernel,
    out_type=jax.ShapeDtypeStruct((B, H), jnp.float32),
    mesh=plsc.VectorSubcoreMesh(core_axis_name="c", subcore_axis_name="s",
                                num_cores=NC, num_subcores=NS),
    scratch_types=[
        pltpu.VMEM((BPW, S), jnp.int32),
        pltpu.VMEM((CHK[2], H), jnp.float32),
        pltpu.VMEM((CHK[2], H), jnp.float32),
        pltpu.VMEM((CHK[2], H), jnp.float32),
        pltpu.VMEM((CHK[2], H), jnp.float32),
        pltpu.VMEM((CHK[2], H), jnp.float32),
        pltpu.VMEM((CHK[2], H), jnp.float32),
        pltpu.VMEM((BPW, H), jnp.float32),
        pltpu.SemaphoreType.DMA,
        pltpu.SemaphoreType.DMA,
        pltpu.SemaphoreType.DMA,
        pltpu.SemaphoreType.DMA,
        pltpu.SemaphoreType.DMA,
        pltpu.SemaphoreType.DMA,
    ],
    compiler_params=pltpu.CompilerParams(use_tc_tiling_on_sc=True),
)
def _pool_sc(idx_hbm, g_hbm, out_hbm, idx_v, buf0, buf1, buf2, buf3, buf4,
             buf5, out_v, sem0, sem1, sem2, sem3, sem4, sem5):
    _pool_sc_kernel(idx_hbm, g_hbm, out_hbm, idx_v, buf0, buf1, buf2, buf3,
                    buf4, buf5, out_v, sem0, sem1, sem2, sem3, sem4, sem5)


def _epilogue_body(p_ref, b1_ref, w2_ref, b2_ref, o_ref):
    h = jnp.maximum(p_ref[...] * (1.0 / S) + b1_ref[...], 0.0)
    o = jnp.dot(h, w2_ref[...], preferred_element_type=jnp.float32)
    o_ref[...] = o + b2_ref[...]


def _epilogue_tc(pooled, b1, w2, b2):
    return pl.pallas_call(
        _epilogue_body,
        out_shape=jax.ShapeDtypeStruct((B, C), jnp.float32),
    )(pooled, b1.reshape(1, H), w2, b2.reshape(1, C))


def kernel(reports, table, W1, b1, W2, b2):
    g = _expand_tc(table.T, W1)
    pooled = _pool_sc(reports, g)
    return _epilogue_tc(pooled, b1, W2, b2)
